# baseline (speedup 1.0000x reference)
import functools

import jax
import jax.numpy as jnp
from jax import lax
from jax.experimental import pallas as pl
from jax.experimental.pallas import tpu as pltpu
from jax.experimental.pallas import tpu_sc as plsc

L = 16
NC = 2
NS = 16
NW = NC * NS
B = 16384
D = 64
BPW = B // NW
CHUNK = 128
NCH = BPW // CHUNK
CH = 32
NCH2 = BPW // CH
TB = 8
NBLK = 125000


def _bias_gather(item_ids, bias1d):
    mesh = plsc.VectorSubcoreMesh(core_axis_name="c", subcore_axis_name="s")

    @functools.partial(
        pl.kernel,
        out_type=jax.ShapeDtypeStruct((B,), jnp.float32),
        mesh=mesh,
        compiler_params=pltpu.CompilerParams(
            needs_layout_passes=False, use_tc_tiling_on_sc=False),
        scratch_types=[
            pltpu.VMEM((NCH, CHUNK), jnp.int32),
            pltpu.VMEM((BPW,), jnp.float32),
            pltpu.SemaphoreType.DMA,
        ],
    )
    def run(iids_hbm, ib_hbm, out_hbm, iidx, brows, sem):
        wid = lax.axis_index("s") * NC + lax.axis_index("c")
        base = wid * BPW
        for c in range(NCH):
            pltpu.sync_copy(iids_hbm.at[pl.ds(base + c * CHUNK, CHUNK)],
                            iidx.at[c])
        copies = [
            pltpu.async_copy(ib_hbm.at[iidx.at[c]],
                             brows.at[pl.ds(c * CHUNK, CHUNK)], sem)
            for c in range(NCH)
        ]
        for cp in copies:
            cp.wait()
        pltpu.sync_copy(brows, out_hbm.at[pl.ds(base, BPW)])

    return run(item_ids, bias1d)


def _dot_kernel(user_ids, item_ids, uf3, if3, bvec):
    mesh = plsc.VectorSubcoreMesh(core_axis_name="c", subcore_axis_name="s")

    @functools.partial(
        pl.kernel,
        out_type=jax.ShapeDtypeStruct((B,), jnp.float32),
        mesh=mesh,
        compiler_params=pltpu.CompilerParams(
            needs_layout_passes=False, use_tc_tiling_on_sc=False),
        scratch_types=[
            pltpu.VMEM((NCH2, CH), jnp.int32),
            pltpu.VMEM((NCH2, CH), jnp.int32),
            pltpu.VMEM((NCH2, CH), jnp.int32),
            pltpu.VMEM((NCH2, CH), jnp.int32),
            pltpu.VMEM((2, CH, TB, D), jnp.float32),
            pltpu.VMEM((2, CH, TB, D), jnp.float32),
            pltpu.VMEM((BPW,), jnp.float32),
            pltpu.VMEM((BPW,), jnp.float32),
            pltpu.SemaphoreType.DMA,
            pltpu.SemaphoreType.DMA,
        ],
    )
    def run(uids_hbm, iids_hbm, uf_hbm, if_hbm, bv_hbm, out_hbm,
            uidx, iidx, ublk, iblk, ubufs, ibufs, bv, outv, sem0, sem1):
        wid = lax.axis_index("s") * NC + lax.axis_index("c")
        base = wid * BPW

        pltpu.sync_copy(bv_hbm.at[pl.ds(base, BPW)], bv)
        for c in range(NCH2):
            pltpu.sync_copy(uids_hbm.at[pl.ds(base + c * CH, CH)],
                            uidx.at[c])
            pltpu.sync_copy(iids_hbm.at[pl.ds(base + c * CH, CH)],
                            iidx.at[c])
        for c in range(NCH2):
            for g in range(CH // L):
                sl = pl.ds(g * L, L)
                ublk[c, sl] = jax.lax.shift_right_logical(uidx[c, sl], 3)
                iblk[c, sl] = jax.lax.shift_right_logical(iidx[c, sl], 3)

        sems = (sem0, sem1)

        def fire(c, buf):
            pltpu.async_copy(uf_hbm.at[ublk.at[c]], ubufs.at[buf], sems[buf])
            pltpu.async_copy(if_hbm.at[iblk.at[c]], ibufs.at[buf], sems[buf])

        def drain(buf):
            pltpu.make_async_copy(
                uf_hbm.at[pl.ds(0, CH)], ubufs.at[buf], sems[buf]).wait()
            pltpu.make_async_copy(
                if_hbm.at[pl.ds(0, CH)], ibufs.at[buf], sems[buf]).wait()

        def compute(c, buf):
            ublocks = ubufs.at[buf]
            iblocks = ibufs.at[buf]
            for g in range(CH // L):
                sl = pl.ds(g * L, L)
                jvec = lax.iota(jnp.int32, L) + g * L
                urow = jnp.bitwise_and(uidx[c, sl], 7)
                irow = jnp.bitwise_and(iidx[c, sl], 7)
                acc = bv[pl.ds(c * CH + g * L, L)]
                for d in range(D):
                    col = jnp.full((L,), d, jnp.int32)
                    u = plsc.load_gather(ublocks, [jvec, urow, col])
                    it = plsc.load_gather(iblocks, [jvec, irow, col])
                    acc = acc + u * it
                outv[pl.ds(c * CH + g * L, L)] = acc

        fire(0, 0)

        def pair_body(t, carry):
            c0 = t * 2
            c1 = c0 + 1
            fire(c1, 1)
            drain(0)
            compute(c0, 0)

            @pl.when(c1 + 1 < NCH2)
            def _():
                fire(c1 + 1, 0)

            drain(1)
            compute(c1, 1)
            return carry

        lax.fori_loop(0, NCH2 // 2, pair_body, 0)
        pltpu.sync_copy(outv, out_hbm.at[pl.ds(base, BPW)])

    return run(user_ids, item_ids, uf3, if3, bvec)


def kernel(user_ids, item_ids, user_factors, item_factors, item_bias):
    bvec = _bias_gather(item_ids, item_bias.reshape(-1))
    uf3 = user_factors.reshape(NBLK, TB, D)
    if3 = item_factors.reshape(NBLK, TB, D)
    return _dot_kernel(user_ids, item_ids, uf3, if3, bvec)


# speedup vs baseline: 2.0780x; 2.0780x over previous
import functools

import jax
import jax.numpy as jnp
from jax import lax
from jax.experimental import pallas as pl
from jax.experimental.pallas import tpu as pltpu
from jax.experimental.pallas import tpu_sc as plsc

L = 16
NC = 2
NS = 16
NW = NC * NS
B = 16384
D = 64
BPW = B // NW
CHUNK = 128
NCH = BPW // CHUNK
CH = 32
NCH2 = BPW // CH
TB = 8
NBLK = 125000


def _bias_gather(item_ids, bias1d):
    mesh = plsc.VectorSubcoreMesh(core_axis_name="c", subcore_axis_name="s")

    @functools.partial(
        pl.kernel,
        out_type=jax.ShapeDtypeStruct((B,), jnp.float32),
        mesh=mesh,
        compiler_params=pltpu.CompilerParams(
            needs_layout_passes=False, use_tc_tiling_on_sc=False),
        scratch_types=[
            pltpu.VMEM((NCH, CHUNK), jnp.int32),
            pltpu.VMEM((BPW,), jnp.float32),
            pltpu.SemaphoreType.DMA,
        ],
    )
    def run(iids_hbm, ib_hbm, out_hbm, iidx, brows, sem):
        wid = lax.axis_index("s") * NC + lax.axis_index("c")
        base = wid * BPW
        for c in range(NCH):
            pltpu.sync_copy(iids_hbm.at[pl.ds(base + c * CHUNK, CHUNK)],
                            iidx.at[c])
        copies = [
            pltpu.async_copy(ib_hbm.at[iidx.at[c]],
                             brows.at[pl.ds(c * CHUNK, CHUNK)], sem)
            for c in range(NCH)
        ]
        for cp in copies:
            cp.wait()
        pltpu.sync_copy(brows, out_hbm.at[pl.ds(base, BPW)])

    return run(item_ids, bias1d)


def _dot_kernel(user_ids, item_ids, uf3, if3, bvec):
    mesh = plsc.VectorSubcoreMesh(core_axis_name="c", subcore_axis_name="s")

    @functools.partial(
        pl.kernel,
        out_type=jax.ShapeDtypeStruct((B,), jnp.float32),
        mesh=mesh,
        compiler_params=pltpu.CompilerParams(needs_layout_passes=False),
        scratch_types=[
            pltpu.VMEM((NCH2, CH), jnp.int32),
            pltpu.VMEM((NCH2, CH), jnp.int32),
            pltpu.VMEM((CH, TB, D), jnp.float32),
            pltpu.VMEM((CH, TB, D), jnp.float32),
            pltpu.VMEM((BPW,), jnp.float32),
            pltpu.VMEM((BPW,), jnp.float32),
            pltpu.SemaphoreType.DMA,
        ],
    )
    def run(uids_hbm, iids_hbm, uf_hbm, if_hbm, bv_hbm, out_hbm,
            uidx, iidx, ublocks, iblocks, bv, outv, sem):
        wid = lax.axis_index("s") * NC + lax.axis_index("c")
        base = wid * BPW

        pltpu.sync_copy(bv_hbm.at[pl.ds(base, BPW)], bv)
        for c in range(NCH2):
            pltpu.sync_copy(uids_hbm.at[pl.ds(base + c * CH, CH)],
                            uidx.at[c])
            pltpu.sync_copy(iids_hbm.at[pl.ds(base + c * CH, CH)],
                            iidx.at[c])

        def chunk_body(c, carry):
            for g in range(CH // L):
                uvec = jax.lax.shift_right_logical(uidx[c, pl.ds(g * L, L)], 3)
                ivec = jax.lax.shift_right_logical(iidx[c, pl.ds(g * L, L)], 3)
                for j in range(L):
                    r = g * L + j
                    pltpu.make_async_copy(
                        uf_hbm.at[uvec[j]], ublocks.at[r], sem).start()
                    pltpu.make_async_copy(
                        if_hbm.at[ivec[j]], iblocks.at[r], sem).start()
            pltpu.make_async_copy(
                uf_hbm.at[pl.ds(0, CH)], ublocks, sem).wait()
            pltpu.make_async_copy(
                if_hbm.at[pl.ds(0, CH)], iblocks, sem).wait()

            for g in range(CH // L):
                sl = pl.ds(g * L, L)
                jvec = lax.iota(jnp.int32, L) + g * L
                urow = jnp.bitwise_and(uidx[c, sl], 7)
                irow = jnp.bitwise_and(iidx[c, sl], 7)
                acc = bv[pl.ds(c * CH + g * L, L)]
                for d in range(D):
                    col = jnp.full((L,), d, jnp.int32)
                    u = plsc.load_gather(ublocks, [jvec, urow, col])
                    it = plsc.load_gather(iblocks, [jvec, irow, col])
                    acc = acc + u * it
                outv[pl.ds(c * CH + g * L, L)] = acc
            return carry

        lax.fori_loop(0, NCH2, chunk_body, 0)
        pltpu.sync_copy(outv, out_hbm.at[pl.ds(base, BPW)])

    return run(user_ids, item_ids, uf3, if3, bvec)


def kernel(user_ids, item_ids, user_factors, item_factors, item_bias):
    bvec = _bias_gather(item_ids, item_bias.reshape(-1))
    uf3 = user_factors.reshape(NBLK, TB, D)
    if3 = item_factors.reshape(NBLK, TB, D)
    return _dot_kernel(user_ids, item_ids, uf3, if3, bvec)


# speedup vs baseline: 2.2687x; 1.0918x over previous
import functools

import jax
import jax.numpy as jnp
from jax import lax
from jax.experimental import pallas as pl
from jax.experimental.pallas import tpu as pltpu
from jax.experimental.pallas import tpu_sc as plsc

L = 16
NC = 2
NS = 16
NW = NC * NS
B = 16384
D = 64
BPW = B // NW
CHUNK = 128
NCH = BPW // CHUNK
CH = 128
NCH2 = BPW // CH
TB = 8
NBLK = 125000


def _bias_gather(item_ids, bias1d):
    mesh = plsc.VectorSubcoreMesh(core_axis_name="c", subcore_axis_name="s")

    @functools.partial(
        pl.kernel,
        out_type=jax.ShapeDtypeStruct((B,), jnp.float32),
        mesh=mesh,
        compiler_params=pltpu.CompilerParams(
            needs_layout_passes=False, use_tc_tiling_on_sc=False),
        scratch_types=[
            pltpu.VMEM((NCH, CHUNK), jnp.int32),
            pltpu.VMEM((BPW,), jnp.float32),
            pltpu.SemaphoreType.DMA,
        ],
    )
    def run(iids_hbm, ib_hbm, out_hbm, iidx, brows, sem):
        wid = lax.axis_index("s") * NC + lax.axis_index("c")
        base = wid * BPW
        for c in range(NCH):
            pltpu.sync_copy(iids_hbm.at[pl.ds(base + c * CHUNK, CHUNK)],
                            iidx.at[c])
        copies = [
            pltpu.async_copy(ib_hbm.at[iidx.at[c]],
                             brows.at[pl.ds(c * CHUNK, CHUNK)], sem)
            for c in range(NCH)
        ]
        for cp in copies:
            cp.wait()
        pltpu.sync_copy(brows, out_hbm.at[pl.ds(base, BPW)])

    return run(item_ids, bias1d)


def _dot_kernel(user_ids, item_ids, uf3, if3, bvec):
    mesh = plsc.VectorSubcoreMesh(core_axis_name="c", subcore_axis_name="s")

    @functools.partial(
        pl.kernel,
        out_type=jax.ShapeDtypeStruct((B,), jnp.float32),
        mesh=mesh,
        compiler_params=pltpu.CompilerParams(needs_layout_passes=False),
        scratch_types=[
            pltpu.VMEM((NCH2, CH), jnp.int32),
            pltpu.VMEM((NCH2, CH), jnp.int32),
            pltpu.VMEM((CH // TB, TB, D), jnp.float32),
            pltpu.VMEM((CH // TB, TB, D), jnp.float32),
            pltpu.VMEM((BPW,), jnp.float32),
            pltpu.VMEM((BPW,), jnp.float32),
            pltpu.SemaphoreType.DMA,
        ],
    )
    def run(uids_hbm, iids_hbm, uf_hbm, if_hbm, bv_hbm, out_hbm,
            uidx, iidx, ublocks, iblocks, bv, outv, sem):
        wid = lax.axis_index("s") * NC + lax.axis_index("c")
        base = wid * BPW

        pltpu.sync_copy(bv_hbm.at[pl.ds(base, BPW)], bv)
        for c in range(NCH2):
            pltpu.sync_copy(uids_hbm.at[pl.ds(base + c * CH, CH)],
                            uidx.at[c])
            pltpu.sync_copy(iids_hbm.at[pl.ds(base + c * CH, CH)],
                            iidx.at[c])

        def chunk_body(c, carry):
            for g in range(CH // L):
                uvec = uidx[c, pl.ds(g * L, L)]
                ivec = iidx[c, pl.ds(g * L, L)]
                ublkv = jax.lax.shift_right_logical(uvec, 3)
                iblkv = jax.lax.shift_right_logical(ivec, 3)
                usubv = jnp.bitwise_and(uvec, 7)
                isubv = jnp.bitwise_and(ivec, 7)
                for j in range(L):
                    r = g * L + j
                    pltpu.make_async_copy(
                        uf_hbm.at[ublkv[j], usubv[j]],
                        ublocks.at[r // TB, r % TB], sem).start()
                    pltpu.make_async_copy(
                        if_hbm.at[iblkv[j], isubv[j]],
                        iblocks.at[r // TB, r % TB], sem).start()
            pltpu.make_async_copy(
                uf_hbm.at[pl.ds(0, CH // TB)], ublocks, sem).wait()
            pltpu.make_async_copy(
                if_hbm.at[pl.ds(0, CH // TB)], iblocks, sem).wait()

            for g in range(CH // L):
                jvec = lax.iota(jnp.int32, L) + g * L
                j8 = jax.lax.shift_right_logical(jvec, 3)
                jsub = jnp.bitwise_and(jvec, 7)
                acc = bv[pl.ds(c * CH + g * L, L)]
                for d in range(D):
                    col = jnp.full((L,), d, jnp.int32)
                    u = plsc.load_gather(ublocks, [j8, jsub, col])
                    it = plsc.load_gather(iblocks, [j8, jsub, col])
                    acc = acc + u * it
                outv[pl.ds(c * CH + g * L, L)] = acc
            return carry

        lax.fori_loop(0, NCH2, chunk_body, 0)
        pltpu.sync_copy(outv, out_hbm.at[pl.ds(base, BPW)])

    return run(user_ids, item_ids, uf3, if3, bvec)


def kernel(user_ids, item_ids, user_factors, item_factors, item_bias):
    bvec = _bias_gather(item_ids, item_bias.reshape(-1))
    uf3 = user_factors.reshape(NBLK, TB, D)
    if3 = item_factors.reshape(NBLK, TB, D)
    return _dot_kernel(user_ids, item_ids, uf3, if3, bvec)
